# baseline (speedup 1.0000x reference)
import jax
import jax.numpy as jnp
from jax.experimental import pallas as pl
from jax.experimental.pallas import tpu as pltpu


def _pick_rows(n, candidates):
    for r in candidates:
        if n % r == 0:
            return r
    return n


def _cast_l0_kernel(features_ref, adj_ref, w0_ref, w1_ref, b0_ref,
                    adjb_ref, sup1_ref, sup0_s, *, n_pad):
    r = pl.program_id(0)

    @pl.when(r == 0)
    def _():
        sup0_s[...] = jnp.dot(features_ref[...], w0_ref[...],
                              preferred_element_type=jnp.float32
                              ).astype(jnp.bfloat16)

    ab = adj_ref[...].astype(jnp.bfloat16)
    if n_pad > ab.shape[1]:
        adjb_ref[...] = jnp.concatenate(
            [ab, jnp.zeros((ab.shape[0], n_pad - ab.shape[1]), jnp.bfloat16)],
            axis=1)
    else:
        adjb_ref[...] = ab
    x = jnp.maximum(jnp.dot(ab, sup0_s[...],
                            preferred_element_type=jnp.float32)
                    + b0_ref[...], 0.0)
    sup1_ref[...] = jnp.dot(x, w1_ref[...],
                            preferred_element_type=jnp.float32
                            ).astype(jnp.bfloat16)


def _stack_kernel(features_ref, sup1_ref, adjb_ref, wn_ref, b_ref,
                  out_ref, sup_s, *, rows, n_layers, h, n):
    l = pl.program_id(0) + 1
    r = pl.program_id(1)
    last = n_layers - 1
    p = jax.lax.rem(l, 2)
    rs = pl.ds(r * rows, rows)
    n_pad = sup_s.shape[1]

    @pl.when(jnp.logical_and(l == 1, r == 0))
    def _():
        sup_s[1, :n, :] = sup1_ref[...]
        if n_pad > n:
            sup_s[0, n:, :] = jnp.zeros((n_pad - n, sup_s.shape[2]),
                                        jnp.bfloat16)
            sup_s[1, n:, :] = jnp.zeros((n_pad - n, sup_s.shape[2]),
                                        jnp.bfloat16)

    val = jnp.dot(adjb_ref[...], sup_s[p],
                  preferred_element_type=jnp.float32) + b_ref[0]
    x = jnp.maximum(val, 0.0)


    @pl.when(jnp.logical_and(jnp.logical_or(p == 1, l == last - 1),
                             l < last))
    def _():
        prev = jnp.where(l == 1, features_ref[rs, :h], out_ref[rs, h:])
        fnew = (prev + x) * 0.5
        out_ref[rs, h:] = fnew
        sup_s[1 - p, rs, :] = jnp.dot(fnew, wn_ref[0],
                                      preferred_element_type=jnp.float32
                                      ).astype(jnp.bfloat16)

    @pl.when(jnp.logical_and(p == 0, l < last - 1))
    def _():
        sup_s[1 - p, rs, :] = jnp.dot(x, wn_ref[0],
                                      preferred_element_type=jnp.float32
                                      ).astype(jnp.bfloat16)

    @pl.when(l == last)
    def _():
        out_ref[rs, :h] = val


def kernel(features, adj, Ws, bs):
    n, f_in = features.shape
    h = Ws[0].shape[1]
    out_dim = Ws[-1].shape[1]
    n_layers = len(Ws)

    wr = jnp.stack([
        w if w.shape == (h, h) else
        jnp.zeros((h, h), jnp.float32).at[:, :w.shape[1]].set(w)
        for w in Ws[1:]
    ])
    bst = jnp.stack([
        (b if b.shape[0] == h else
         jnp.zeros((h,), jnp.float32).at[:b.shape[0]].set(b)).reshape(1, h)
        for b in bs[1:]
    ])

    n_pad = ((n + 127) // 128) * 128
    rows_a = _pick_rows(n, (400, 200, 100, 80, 40, 16, 8))
    adjb, sup1 = pl.pallas_call(
        lambda *refs: _cast_l0_kernel(*refs, n_pad=n_pad),
        grid=(n // rows_a,),
        in_specs=[
            pl.BlockSpec((n, f_in), lambda r: (0, 0)),
            pl.BlockSpec((rows_a, n), lambda r: (r, 0)),
            pl.BlockSpec((f_in, h), lambda r: (0, 0)),
            pl.BlockSpec((h, h), lambda r: (0, 0)),
            pl.BlockSpec((1, h), lambda r: (0, 0)),
        ],
        out_specs=[
            pl.BlockSpec((rows_a, n_pad), lambda r: (r, 0)),
            pl.BlockSpec((rows_a, h), lambda r: (r, 0)),
        ],
        out_shape=[
            jax.ShapeDtypeStruct((n, n_pad), jnp.bfloat16),
            jax.ShapeDtypeStruct((n, h), jnp.bfloat16),
        ],
        scratch_shapes=[pltpu.VMEM((n, h), jnp.bfloat16)],
    )(features, adj, Ws[0], Ws[1], bs[0].reshape(1, h))

    rows = _pick_rows(n, (1000, 400, 200, 100, 80, 40, 16, 8))
    grid = (n_layers - 1, n // rows)
    big_out = pl.pallas_call(
        lambda *refs: _stack_kernel(*refs, rows=rows, n_layers=n_layers,
                                    h=h, n=n),
        grid=grid,
        in_specs=[
            pl.BlockSpec((n, f_in), lambda l, r: (0, 0)),
            pl.BlockSpec((n, h), lambda l, r: (0, 0)),
            pl.BlockSpec((rows, n_pad), lambda l, r: (r, 0)),
            pl.BlockSpec((1, h, h), lambda l, r: (jnp.minimum(l + 1, 12), 0, 0)),
            pl.BlockSpec((1, 1, h), lambda l, r: (l, 0, 0)),
        ],
        out_specs=pl.BlockSpec((n, 2 * h), lambda l, r: (0, 0)),
        out_shape=jax.ShapeDtypeStruct((n, 2 * h), jnp.float32),
        scratch_shapes=[
            pltpu.VMEM((2, n_pad, h), jnp.bfloat16),
        ],
    )(features, sup1, adjb, wr, bst.reshape(n_layers - 1, 1, h))
    return (big_out[:, :out_dim], big_out[:, h:])


# speedup vs baseline: 1.0082x; 1.0082x over previous
import jax
import jax.numpy as jnp
from jax.experimental import pallas as pl
from jax.experimental.pallas import tpu as pltpu


def _pick_rows(n, candidates):
    for r in candidates:
        if n % r == 0:
            return r
    return n


def _cast_l0_kernel(features_ref, adj_ref, w0_ref, w1_ref, b0_ref,
                    adjb_ref, sup1_ref, sup0_s, *, n_pad):
    r = pl.program_id(0)

    @pl.when(r == 0)
    def _():
        sup0_s[...] = jnp.dot(features_ref[...], w0_ref[...],
                              preferred_element_type=jnp.float32
                              ).astype(jnp.bfloat16)

    ab = adj_ref[...].astype(jnp.bfloat16)
    if n_pad > ab.shape[1]:
        adjb_ref[...] = jnp.concatenate(
            [ab, jnp.zeros((ab.shape[0], n_pad - ab.shape[1]), jnp.bfloat16)],
            axis=1)
    else:
        adjb_ref[...] = ab
    x = jnp.maximum(jnp.dot(ab, sup0_s[...],
                            preferred_element_type=jnp.float32)
                    + b0_ref[...], 0.0)
    sup1_ref[...] = jnp.dot(x, w1_ref[...],
                            preferred_element_type=jnp.float32
                            ).astype(jnp.bfloat16)


def _stack_kernel(features_ref, sup1_ref, adjb_hbm, wn_ref, b_ref,
                  out_ref, abuf, sup_s, dma_sems,
                  *, rows, n_layers, h, n, nblk, nbuf):
    lb = pl.program_id(0)
    l = lb + 1
    r = pl.program_id(1)
    last = n_layers - 1
    p = jax.lax.rem(l, 2)
    rs = pl.ds(r * rows, rows)
    n_pad = sup_s.shape[1]
    idx = lb * nblk + r
    total = (n_layers - 1) * nblk

    def _fetch(i):
        blk = jax.lax.rem(i, nblk)
        buf = jax.lax.rem(i, nbuf)
        return pltpu.make_async_copy(
            adjb_hbm.at[pl.ds(blk * rows, rows), :],
            abuf.at[buf],
            dma_sems.at[buf],
        )

    @pl.when(idx == 0)
    def _():
        sup_s[1, :n, :] = sup1_ref[...]
        if n_pad > n:
            sup_s[0, n:, :] = jnp.zeros((n_pad - n, sup_s.shape[2]),
                                        jnp.bfloat16)
            sup_s[1, n:, :] = jnp.zeros((n_pad - n, sup_s.shape[2]),
                                        jnp.bfloat16)
        for j in range(nbuf - 1):
            _fetch(jnp.int32(j)).start()

    @pl.when(idx + nbuf - 1 < total)
    def _():
        _fetch(idx + nbuf - 1).start()

    _fetch(idx).wait()
    buf = jax.lax.rem(idx, nbuf)

    val = jnp.dot(abuf[buf], sup_s[p],
                  preferred_element_type=jnp.float32) + b_ref[0]
    x = jnp.maximum(val, 0.0)


    @pl.when(jnp.logical_and(jnp.logical_or(p == 1, l == last - 1),
                             l < last))
    def _():
        prev = jnp.where(l == 1, features_ref[rs, :h], out_ref[rs, h:])
        fnew = (prev + x) * 0.5
        out_ref[rs, h:] = fnew
        sup_s[1 - p, rs, :] = jnp.dot(fnew, wn_ref[0],
                                      preferred_element_type=jnp.float32
                                      ).astype(jnp.bfloat16)

    @pl.when(jnp.logical_and(p == 0, l < last - 1))
    def _():
        sup_s[1 - p, rs, :] = jnp.dot(x, wn_ref[0],
                                      preferred_element_type=jnp.float32
                                      ).astype(jnp.bfloat16)

    @pl.when(l == last)
    def _():
        out_ref[rs, :h] = val


def kernel(features, adj, Ws, bs):
    n, f_in = features.shape
    h = Ws[0].shape[1]
    out_dim = Ws[-1].shape[1]
    n_layers = len(Ws)

    wr = jnp.stack([
        w if w.shape == (h, h) else
        jnp.zeros((h, h), jnp.float32).at[:, :w.shape[1]].set(w)
        for w in Ws[1:]
    ])
    bst = jnp.stack([
        (b if b.shape[0] == h else
         jnp.zeros((h,), jnp.float32).at[:b.shape[0]].set(b)).reshape(1, h)
        for b in bs[1:]
    ])

    n_pad = ((n + 127) // 128) * 128
    rows_a = _pick_rows(n, (400, 200, 100, 80, 40, 16, 8))
    adjb, sup1 = pl.pallas_call(
        lambda *refs: _cast_l0_kernel(*refs, n_pad=n_pad),
        grid=(n // rows_a,),
        in_specs=[
            pl.BlockSpec((n, f_in), lambda r: (0, 0)),
            pl.BlockSpec((rows_a, n), lambda r: (r, 0)),
            pl.BlockSpec((f_in, h), lambda r: (0, 0)),
            pl.BlockSpec((h, h), lambda r: (0, 0)),
            pl.BlockSpec((1, h), lambda r: (0, 0)),
        ],
        out_specs=[
            pl.BlockSpec((rows_a, n_pad), lambda r: (r, 0)),
            pl.BlockSpec((rows_a, h), lambda r: (r, 0)),
        ],
        out_shape=[
            jax.ShapeDtypeStruct((n, n_pad), jnp.bfloat16),
            jax.ShapeDtypeStruct((n, h), jnp.bfloat16),
        ],
        scratch_shapes=[pltpu.VMEM((n, h), jnp.bfloat16)],
    )(features, adj, Ws[0], Ws[1], bs[0].reshape(1, h))

    rows = _pick_rows(n, (400, 200, 100, 80, 40, 16, 8))
    nblk = n // rows
    nbuf = 4
    grid = (n_layers - 1, nblk)
    big_out = pl.pallas_call(
        lambda *refs: _stack_kernel(*refs, rows=rows, n_layers=n_layers,
                                    h=h, n=n, nblk=nblk, nbuf=nbuf),
        grid=grid,
        in_specs=[
            pl.BlockSpec((n, f_in), lambda l, r: (0, 0)),
            pl.BlockSpec((n, h), lambda l, r: (0, 0)),
            pl.BlockSpec(memory_space=pltpu.MemorySpace.HBM),
            pl.BlockSpec((1, h, h), lambda l, r: (jnp.minimum(l + 1, 12), 0, 0)),
            pl.BlockSpec((1, 1, h), lambda l, r: (l, 0, 0)),
        ],
        out_specs=pl.BlockSpec((n, 2 * h), lambda l, r: (0, 0)),
        out_shape=jax.ShapeDtypeStruct((n, 2 * h), jnp.float32),
        scratch_shapes=[
            pltpu.VMEM((nbuf, rows, n_pad), jnp.bfloat16),
            pltpu.VMEM((2, n_pad, h), jnp.bfloat16),
            pltpu.SemaphoreType.DMA((nbuf,)),
        ],
    )(features, sup1, adjb, wr, bst.reshape(n_layers - 1, 1, h))
    return (big_out[:, :out_dim], big_out[:, h:])
